# Initial kernel scaffold; baseline (speedup 1.0000x reference)
import jax
import jax.numpy as jnp
from jax.experimental import pallas as pl


def kernel(x, token_table, pos_table):
    raise NotImplementedError("write your pallas kernel here")


# speedup vs baseline: 1.3612x; 1.3612x over previous
import functools

import jax
import jax.numpy as jnp
from jax import lax
from jax.experimental import pallas as pl
from jax.experimental.pallas import tpu as pltpu
from jax.experimental.pallas import tpu_sc as plsc

_VOCAB = 1000000
_MAXLEN = 200
_EMBED = 32
_BATCH = 4096

_NC = 2
_NS = 16
_NW = _NC * _NS

_N = _BATCH * _MAXLEN
_PER_W = _N // _NW
_TILE = 4
_C = _TILE * _MAXLEN
_NCHUNK = _PER_W // _C


def _emb_kernel(x_hbm, tok_hbm, pos_hbm, out_hbm, idx_v, rows_v, pos_v, sem):
    wid = lax.axis_index("s") * _NC + lax.axis_index("c")
    base = wid * _PER_W

    for j in range(_TILE):
        pltpu.sync_copy(pos_hbm, pos_v.at[pl.ds(j * _MAXLEN, _MAXLEN)])

    def chunk_body(cix, carry):
        off = base + cix * _C
        pltpu.sync_copy(x_hbm.at[pl.ds(off, _C)], idx_v)
        pltpu.async_copy(tok_hbm.at[idx_v], rows_v, sem).wait()

        def add_body(i, carry2):
            p0 = pos_v[i, pl.ds(0, 16)]
            p1 = pos_v[i, pl.ds(16, 16)]
            plsc.addupdate(rows_v.at[i, pl.ds(0, 16)], p0)
            plsc.addupdate(rows_v.at[i, pl.ds(16, 16)], p1)
            return carry2

        lax.fori_loop(0, _C, add_body, 0, unroll=4)
        pltpu.sync_copy(rows_v, out_hbm.at[pl.ds(off, _C)])
        return carry

    lax.fori_loop(0, _NCHUNK, chunk_body, 0)


@jax.jit
def _run(x_flat, token_table, pos_table):
    mesh = plsc.VectorSubcoreMesh(core_axis_name="c", subcore_axis_name="s")
    f = functools.partial(
        pl.kernel,
        mesh=mesh,
        out_type=jax.ShapeDtypeStruct((_N, _EMBED), jnp.float32),
        scratch_types=[
            pltpu.VMEM((_C,), jnp.int32),
            pltpu.VMEM((_C, _EMBED), jnp.float32),
            pltpu.VMEM((_C, _EMBED), jnp.float32),
            pltpu.SemaphoreType.DMA,
        ],
        compiler_params=pltpu.CompilerParams(use_tc_tiling_on_sc=False),
    )(_emb_kernel)
    return f(x_flat, token_table, pos_table)


def kernel(x, token_table, pos_table):
    x_flat = x.reshape(-1).astype(jnp.int32)
    out = _run(x_flat, token_table, pos_table)
    return out.reshape(_BATCH, _MAXLEN, _EMBED)
